# initial kernel scaffold (unmeasured)
import jax
import jax.numpy as jnp
from jax import lax
from jax.experimental import pallas as pl
from jax.experimental.pallas import tpu as pltpu

N_DEV = 4


def _gelu(y):
    c = 0.7978845608028654
    return 0.5 * y * (1.0 + jnp.tanh(c * (y + 0.044715 * y * y * y)))


def kernel(x, w_mat):
    m_per, k = x.shape
    _, n_per = w_mat.shape
    h2 = m_per // 2
    f32 = jnp.float32

    def body(x_hbm, w_hbm, out_hbm, comm, w_vmem, x_vmem, out_vmem,
             send_sems, recv_sems, local_sems):
        my = lax.axis_index("i")
        right = lax.rem(my + 1, N_DEV)
        left = lax.rem(my + N_DEV - 1, N_DEV)

        cp_w = pltpu.make_async_copy(w_hbm, w_vmem, local_sems.at[0])
        cp_w.start()
        cp_x = pltpu.make_async_copy(x_hbm, x_vmem, local_sems.at[1])
        cp_x.start()

        send_r = pltpu.make_async_remote_copy(
            src_ref=x_hbm, dst_ref=comm.at[0],
            send_sem=send_sems.at[0], recv_sem=recv_sems.at[0],
            device_id=(right,), device_id_type=pl.DeviceIdType.MESH)
        send_r.start()
        send_l = pltpu.make_async_remote_copy(
            src_ref=x_hbm, dst_ref=comm.at[1],
            send_sem=send_sems.at[1], recv_sem=recv_sems.at[1],
            device_id=(left,), device_id_type=pl.DeviceIdType.MESH)
        send_l.start()

        def compute_store(origin, st_sem):
            out_vmem[...] = _gelu(
                jnp.dot(x_vmem[...], w_vmem[...], preferred_element_type=f32))
            st = pltpu.make_async_copy(
                out_vmem, out_hbm.at[pl.ds(origin * m_per, m_per), :], st_sem)
            st.start()
            st.wait()

        cp_w.wait()
        cp_x.wait()
        compute_store(my, local_sems.at[2])

        send_r.wait_recv()
        fwd_r = pltpu.make_async_remote_copy(
            src_ref=comm.at[0, pl.ds(0, h2), :],
            dst_ref=comm.at[2, pl.ds(0, h2), :],
            send_sem=send_sems.at[2], recv_sem=recv_sems.at[2],
            device_id=(right,), device_id_type=pl.DeviceIdType.MESH)
        fwd_r.start()
        cp = pltpu.make_async_copy(comm.at[0], x_vmem, local_sems.at[1])
        cp.start()
        cp.wait()
        compute_store(left, local_sems.at[2])

        send_l.wait_recv()
        fwd_l = pltpu.make_async_remote_copy(
            src_ref=comm.at[1, pl.ds(h2, h2), :],
            dst_ref=comm.at[2, pl.ds(h2, h2), :],
            send_sem=send_sems.at[3], recv_sem=recv_sems.at[3],
            device_id=(left,), device_id_type=pl.DeviceIdType.MESH)
        fwd_l.start()
        cp = pltpu.make_async_copy(comm.at[1], x_vmem, local_sems.at[1])
        cp.start()
        cp.wait()
        compute_store(right, local_sems.at[2])

        fwd_r.wait_recv()
        fwd_l.wait_recv()
        cp = pltpu.make_async_copy(comm.at[2], x_vmem, local_sems.at[1])
        cp.start()
        cp.wait()
        compute_store(lax.rem(my + 2, N_DEV), local_sems.at[2])

        send_r.wait_send()
        send_l.wait_send()
        fwd_r.wait_send()
        fwd_l.wait_send()

    return pl.pallas_call(
        body,
        out_shape=jax.ShapeDtypeStruct((N_DEV * m_per, n_per), f32),
        in_specs=[
            pl.BlockSpec(memory_space=pltpu.MemorySpace.ANY),
            pl.BlockSpec(memory_space=pltpu.MemorySpace.ANY),
        ],
        out_specs=pl.BlockSpec(memory_space=pltpu.MemorySpace.ANY),
        scratch_shapes=[
            pltpu.MemorySpace.HBM((3, m_per, k), f32),
            pltpu.VMEM((k, n_per), f32),
            pltpu.VMEM((m_per, k), f32),
            pltpu.VMEM((m_per, n_per), f32),
            pltpu.SemaphoreType.DMA((4,)),
            pltpu.SemaphoreType.DMA((4,)),
            pltpu.SemaphoreType.DMA((3,)),
        ],
    )(x, w_mat)


# baseline (device time: 361209 ns/iter reference)
import jax
import jax.numpy as jnp
from jax import lax
from jax.experimental import pallas as pl
from jax.experimental.pallas import tpu as pltpu

N_DEV = 4


def _gelu(y):
    c = 0.7978845608028654
    return 0.5 * y * (1.0 + jnp.tanh(c * (y + 0.044715 * y * y * y)))


def kernel(x, w_mat):
    m_per, k = x.shape
    _, n_per = w_mat.shape
    h2 = m_per // 2
    f32 = jnp.float32

    def body(x_hbm, w_hbm, out_hbm, comm, w_vmem, x_vmem, out_vmem,
             send_sems, recv_sems, local_sems):
        my = lax.axis_index("i")
        right = lax.rem(my + 1, N_DEV)
        left = lax.rem(my + N_DEV - 1, N_DEV)

        cp_w = pltpu.make_async_copy(w_hbm, w_vmem, local_sems.at[0])
        cp_w.start()
        cp_x = pltpu.make_async_copy(x_hbm, x_vmem, local_sems.at[1])
        cp_x.start()

        send_r = pltpu.make_async_remote_copy(
            src_ref=x_hbm, dst_ref=comm.at[0],
            send_sem=send_sems.at[0], recv_sem=recv_sems.at[0],
            device_id=(right,), device_id_type=pl.DeviceIdType.MESH)
        send_r.start()
        send_l = pltpu.make_async_remote_copy(
            src_ref=x_hbm, dst_ref=comm.at[1],
            send_sem=send_sems.at[1], recv_sem=recv_sems.at[1],
            device_id=(left,), device_id_type=pl.DeviceIdType.MESH)
        send_l.start()

        def compute_store(origin, st_sem):
            out_vmem[...] = _gelu(
                jnp.dot(x_vmem[...], w_vmem[...], preferred_element_type=f32))
            st = pltpu.make_async_copy(
                out_vmem, out_hbm.at[pl.ds(origin * m_per, m_per), :], st_sem)
            st.start()
            st.wait()

        cp_w.wait()
        cp_x.wait()
        compute_store(my, local_sems.at[2])

        send_r.wait_recv()
        fwd_r = pltpu.make_async_remote_copy(
            src_ref=comm.at[0, pl.ds(0, h2), :],
            dst_ref=comm.at[2, pl.ds(0, h2), :],
            send_sem=send_sems.at[2], recv_sem=recv_sems.at[2],
            device_id=(right,), device_id_type=pl.DeviceIdType.MESH)
        fwd_r.start()
        cp = pltpu.make_async_copy(comm.at[0], x_vmem, local_sems.at[1])
        cp.start()
        cp.wait()
        compute_store(left, local_sems.at[2])

        send_l.wait_recv()
        fwd_l = pltpu.make_async_remote_copy(
            src_ref=comm.at[1, pl.ds(h2, h2), :],
            dst_ref=comm.at[2, pl.ds(h2, h2), :],
            send_sem=send_sems.at[3], recv_sem=recv_sems.at[3],
            device_id=(left,), device_id_type=pl.DeviceIdType.MESH)
        fwd_l.start()
        cp = pltpu.make_async_copy(comm.at[1], x_vmem, local_sems.at[1])
        cp.start()
        cp.wait()
        compute_store(right, local_sems.at[2])

        fwd_r.wait_recv()
        fwd_l.wait_recv()
        cp = pltpu.make_async_copy(comm.at[2], x_vmem, local_sems.at[1])
        cp.start()
        cp.wait()
        compute_store(lax.rem(my + 2, N_DEV), local_sems.at[2])

        send_r.wait_send()
        send_l.wait_send()
        fwd_r.wait_send()
        fwd_l.wait_send()

    out, _ = pl.pallas_call(
        body,
        out_shape=(
            jax.ShapeDtypeStruct((N_DEV * m_per, n_per), f32),
            jax.ShapeDtypeStruct((3, m_per, k), f32),
        ),
        in_specs=[
            pl.BlockSpec(memory_space=pl.ANY),
            pl.BlockSpec(memory_space=pl.ANY),
        ],
        out_specs=(
            pl.BlockSpec(memory_space=pl.ANY),
            pl.BlockSpec(memory_space=pl.ANY),
        ),
        scratch_shapes=[
            pltpu.VMEM((k, n_per), f32),
            pltpu.VMEM((m_per, k), f32),
            pltpu.VMEM((m_per, n_per), f32),
            pltpu.SemaphoreType.DMA((4,)),
            pltpu.SemaphoreType.DMA((4,)),
            pltpu.SemaphoreType.DMA((3,)),
        ],
        compiler_params=pltpu.CompilerParams(
            vmem_limit_bytes=64 * 1024 * 1024,
        ),
    )(x, w_mat)
    return out


# device time: 321638 ns/iter; 1.1230x vs baseline; 1.1230x over previous
import jax
import jax.numpy as jnp
from jax import lax
from jax.experimental import pallas as pl
from jax.experimental.pallas import tpu as pltpu

N_DEV = 4


def _gelu(y):
    c = 0.7978845608028654
    return 0.5 * y * (1.0 + jnp.tanh(c * (y + 0.044715 * y * y * y)))


def kernel(x, w_mat):
    m_per, k = x.shape
    _, n_per = w_mat.shape
    q = m_per // 4
    f32 = jnp.float32

    def body(x_hbm, w_hbm, out_hbm, comm, w_vmem, x_vmem, out_vmem,
             send_sems, recv_sems, local_sems):
        my = lax.axis_index("i")
        right = lax.rem(my + 1, N_DEV)
        left = lax.rem(my + N_DEV - 1, N_DEV)

        cp_w = pltpu.make_async_copy(w_hbm, w_vmem, local_sems.at[0])
        cp_w.start()
        cp_x = pltpu.make_async_copy(x_hbm, x_vmem, local_sems.at[1])
        cp_x.start()

        send_r = pltpu.make_async_remote_copy(
            src_ref=x_hbm, dst_ref=comm.at[0],
            send_sem=send_sems.at[0], recv_sem=recv_sems.at[0],
            device_id=(right,), device_id_type=pl.DeviceIdType.MESH)
        send_r.start()
        send_l = pltpu.make_async_remote_copy(
            src_ref=x_hbm, dst_ref=comm.at[1],
            send_sem=send_sems.at[1], recv_sem=recv_sems.at[1],
            device_id=(left,), device_id_type=pl.DeviceIdType.MESH)
        send_l.start()

        def gemm_piece(row0, nrows):
            rows = pl.ds(row0, nrows)
            out_vmem[rows, :] = _gelu(
                jnp.dot(x_vmem[rows, :], w_vmem[...],
                        preferred_element_type=f32))

        def store(origin):
            st = pltpu.make_async_copy(
                out_vmem, out_hbm.at[pl.ds(origin * m_per, m_per), :],
                local_sems.at[3])
            st.start()
            return st

        cp_w.wait()
        cp_x.wait()
        gemm_piece(0, m_per)
        st = store(my)

        send_r.wait_recv()
        fwd_r0 = pltpu.make_async_remote_copy(
            src_ref=comm.at[0, pl.ds(0, q), :],
            dst_ref=comm.at[2, pl.ds(0, q), :],
            send_sem=send_sems.at[2], recv_sem=recv_sems.at[2],
            device_id=(right,), device_id_type=pl.DeviceIdType.MESH)
        fwd_r0.start()
        fwd_r1 = pltpu.make_async_remote_copy(
            src_ref=comm.at[0, pl.ds(q, q), :],
            dst_ref=comm.at[2, pl.ds(q, q), :],
            send_sem=send_sems.at[3], recv_sem=recv_sems.at[3],
            device_id=(right,), device_id_type=pl.DeviceIdType.MESH)
        fwd_r1.start()
        send_l.wait_recv()
        fwd_l0 = pltpu.make_async_remote_copy(
            src_ref=comm.at[1, pl.ds(2 * q, q), :],
            dst_ref=comm.at[2, pl.ds(2 * q, q), :],
            send_sem=send_sems.at[4], recv_sem=recv_sems.at[4],
            device_id=(left,), device_id_type=pl.DeviceIdType.MESH)
        fwd_l0.start()
        fwd_l1 = pltpu.make_async_remote_copy(
            src_ref=comm.at[1, pl.ds(3 * q, q), :],
            dst_ref=comm.at[2, pl.ds(3 * q, q), :],
            send_sem=send_sems.at[5], recv_sem=recv_sems.at[5],
            device_id=(left,), device_id_type=pl.DeviceIdType.MESH)
        fwd_l1.start()

        cp = pltpu.make_async_copy(comm.at[0], x_vmem, local_sems.at[1])
        cp.start()
        cp.wait()
        st.wait()
        gemm_piece(0, m_per)
        st = store(left)

        cp = pltpu.make_async_copy(comm.at[1], x_vmem, local_sems.at[2])
        cp.start()
        cp.wait()
        st.wait()
        gemm_piece(0, m_per)
        st = store(right)

        fwd_r0.wait_recv()
        fwd_l0.wait_recv()
        cp_a = pltpu.make_async_copy(
            comm.at[2, pl.ds(0, q), :], x_vmem.at[pl.ds(0, q), :],
            local_sems.at[1])
        cp_a.start()
        cp_b = pltpu.make_async_copy(
            comm.at[2, pl.ds(2 * q, q), :], x_vmem.at[pl.ds(2 * q, q), :],
            local_sems.at[2])
        cp_b.start()
        cp_a.wait()
        cp_b.wait()
        st.wait()
        gemm_piece(0, q)
        gemm_piece(2 * q, q)

        fwd_r1.wait_recv()
        fwd_l1.wait_recv()
        cp_a = pltpu.make_async_copy(
            comm.at[2, pl.ds(q, q), :], x_vmem.at[pl.ds(q, q), :],
            local_sems.at[1])
        cp_a.start()
        cp_b = pltpu.make_async_copy(
            comm.at[2, pl.ds(3 * q, q), :], x_vmem.at[pl.ds(3 * q, q), :],
            local_sems.at[2])
        cp_b.start()
        cp_a.wait()
        cp_b.wait()
        gemm_piece(q, q)
        gemm_piece(3 * q, q)
        st = store(lax.rem(my + 2, N_DEV))

        send_r.wait_send()
        send_l.wait_send()
        fwd_r0.wait_send()
        fwd_r1.wait_send()
        fwd_l0.wait_send()
        fwd_l1.wait_send()
        st.wait()

    out, _ = pl.pallas_call(
        body,
        out_shape=(
            jax.ShapeDtypeStruct((N_DEV * m_per, n_per), f32),
            jax.ShapeDtypeStruct((3, m_per, k), f32),
        ),
        in_specs=[
            pl.BlockSpec(memory_space=pl.ANY),
            pl.BlockSpec(memory_space=pl.ANY),
        ],
        out_specs=(
            pl.BlockSpec(memory_space=pl.ANY),
            pl.BlockSpec(memory_space=pl.ANY),
        ),
        scratch_shapes=[
            pltpu.VMEM((k, n_per), f32),
            pltpu.VMEM((m_per, k), f32),
            pltpu.VMEM((m_per, n_per), f32),
            pltpu.SemaphoreType.DMA((6,)),
            pltpu.SemaphoreType.DMA((6,)),
            pltpu.SemaphoreType.DMA((4,)),
        ],
        compiler_params=pltpu.CompilerParams(
            vmem_limit_bytes=64 * 1024 * 1024,
        ),
    )(x, w_mat)
    return out


# device time: 319181 ns/iter; 1.1317x vs baseline; 1.0077x over previous
import jax
import jax.numpy as jnp
from jax import lax
from jax.experimental import pallas as pl
from jax.experimental.pallas import tpu as pltpu

import os as _os
jax.config.update(
    "jax_compilation_cache_dir",
    _os.path.join(_os.path.dirname(_os.path.abspath(__file__)), "jax_cache"),
)
jax.config.update("jax_persistent_cache_min_compile_time_secs", 0)

N_DEV = 4


def _gelu(y):
    c = 0.7978845608028654
    return 0.5 * y * (1.0 + jnp.tanh(c * (y + 0.044715 * y * y * y)))


def kernel(x, w_mat):
    m_per, k = x.shape
    _, n_per = w_mat.shape
    P = m_per // 4
    f32 = jnp.float32

    def body(x_hbm, w_hbm, out_hbm, comm, w_vmem, xbuf, obuf,
             send_sems, recv_sems, cp_sems, st_sems):
        my = lax.axis_index("i")
        right = lax.rem(my + 1, N_DEV)
        left = lax.rem(my + N_DEV - 1, N_DEV)

        cp_w = pltpu.make_async_copy(w_hbm, w_vmem, cp_sems.at[2])
        cp_w.start()

        hop1_r = []
        hop1_l = []
        for kk in range(4):
            rows = pl.ds(kk * P, P)
            d = pltpu.make_async_remote_copy(
                src_ref=x_hbm.at[rows, :], dst_ref=comm.at[0, rows, :],
                send_sem=send_sems.at[kk], recv_sem=recv_sems.at[kk],
                device_id=(right,), device_id_type=pl.DeviceIdType.MESH)
            d.start()
            hop1_r.append(d)
            d = pltpu.make_async_remote_copy(
                src_ref=x_hbm.at[rows, :], dst_ref=comm.at[1, rows, :],
                send_sem=send_sems.at[4 + kk], recv_sem=recv_sems.at[4 + kk],
                device_id=(left,), device_id_type=pl.DeviceIdType.MESH)
            d.start()
            hop1_l.append(d)

        fwd = [None] * 4

        def make_fwd(j):
            src_slot, dst_dev = (0, right) if j < 2 else (1, left)
            rows = pl.ds(j * P, P)
            d = pltpu.make_async_remote_copy(
                src_ref=comm.at[src_slot, rows, :],
                dst_ref=comm.at[2, rows, :],
                send_sem=send_sems.at[8 + j], recv_sem=recv_sems.at[8 + j],
                device_id=(dst_dev,), device_id_type=pl.DeviceIdType.MESH)
            d.start()
            fwd[j] = d

        schedule = []
        for kk in range(4):
            schedule.append((None, x_hbm.at[pl.ds(kk * P, P), :],
                             my * m_per + kk * P, None))
        for kk in range(4):
            post_l = (lambda j=kk: make_fwd(j)) if kk < 2 else None
            post_r = (lambda j=kk: make_fwd(j)) if kk >= 2 else None
            schedule.append((hop1_r[kk].wait_recv,
                             comm.at[0, pl.ds(kk * P, P), :],
                             left * m_per + kk * P, post_l))
            schedule.append((hop1_l[kk].wait_recv,
                             comm.at[1, pl.ds(kk * P, P), :],
                             right * m_per + kk * P, post_r))
        opp = lax.rem(my + 2, N_DEV)
        for j in (0, 2, 1, 3):
            schedule.append(((lambda j=j: fwd[j].wait_recv()),
                             comm.at[2, pl.ds(j * P, P), :],
                             opp * m_per + j * P, None))

        st_desc = [None, None]

        def do_gemm(cp, slot, row):
            cp.wait()
            if st_desc[slot] is not None:
                st_desc[slot].wait()
            obuf[slot, :, :] = _gelu(
                jnp.dot(xbuf[slot, :, :], w_vmem[...],
                        preferred_element_type=f32))
            st = pltpu.make_async_copy(
                obuf.at[slot], out_hbm.at[pl.ds(row, P), :],
                st_sems.at[slot])
            st.start()
            st_desc[slot] = st

        cp_w.wait()
        prev = None
        for i, (wait_fn, src, row, post) in enumerate(schedule):
            slot = i % 2
            if wait_fn is not None:
                wait_fn()
            cp = pltpu.make_async_copy(src, xbuf.at[slot], cp_sems.at[slot])
            cp.start()
            if post is not None:
                post()
            if prev is not None:
                do_gemm(*prev)
            prev = (cp, slot, row)
        do_gemm(*prev)

        for d in hop1_r + hop1_l + fwd:
            d.wait_send()
        st_desc[0].wait()
        st_desc[1].wait()

    out, _ = pl.pallas_call(
        body,
        out_shape=(
            jax.ShapeDtypeStruct((N_DEV * m_per, n_per), f32),
            jax.ShapeDtypeStruct((3, m_per, k), f32),
        ),
        in_specs=[
            pl.BlockSpec(memory_space=pl.ANY),
            pl.BlockSpec(memory_space=pl.ANY),
        ],
        out_specs=(
            pl.BlockSpec(memory_space=pl.ANY),
            pl.BlockSpec(memory_space=pl.ANY),
        ),
        scratch_shapes=[
            pltpu.VMEM((k, n_per), f32),
            pltpu.VMEM((2, P, k), f32),
            pltpu.VMEM((2, P, n_per), f32),
            pltpu.SemaphoreType.DMA((12,)),
            pltpu.SemaphoreType.DMA((12,)),
            pltpu.SemaphoreType.DMA((3,)),
            pltpu.SemaphoreType.DMA((2,)),
        ],
        compiler_params=pltpu.CompilerParams(
            vmem_limit_bytes=64 * 1024 * 1024,
        ),
    )(x, w_mat)
    return out


# device time: 188395 ns/iter; 1.9173x vs baseline; 1.6942x over previous
import jax
import jax.numpy as jnp
from jax import lax
from jax.experimental import pallas as pl
from jax.experimental.pallas import tpu as pltpu

import os as _os
jax.config.update(
    "jax_compilation_cache_dir",
    _os.path.join(_os.path.dirname(_os.path.abspath(__file__)), "jax_cache"),
)
jax.config.update("jax_persistent_cache_min_compile_time_secs", 0)

N_DEV = 4


def _gelu(y):
    c = 0.7978845608028654
    return 0.5 * y * (1.0 + jnp.tanh(c * (y + 0.044715 * y * y * y)))


def kernel(x, w_mat):
    m_per, k = x.shape
    _, n_per = w_mat.shape
    P = m_per // 4
    f32 = jnp.float32
    bf16 = jnp.bfloat16
    KS = 4

    def body(x_hbm, w_hbm, out_hbm, comm, w_bf, xown, xtmp, wtmp, xstage,
             obuf, send_sems, recv_sems, cp_sems, st_sems):
        my = lax.axis_index("i")
        right = lax.rem(my + 1, N_DEV)
        left = lax.rem(my + N_DEV - 1, N_DEV)

        cp = pltpu.make_async_copy(x_hbm, xtmp, cp_sems.at[2])
        cp.start()
        cp.wait()
        xown[...] = xtmp[...].astype(bf16)

        hop1_r = []
        hop1_l = []
        for kk in range(4):
            rows = pl.ds(kk * P, P)
            d = pltpu.make_async_remote_copy(
                src_ref=xown.at[rows, :], dst_ref=comm.at[0, rows, :],
                send_sem=send_sems.at[kk], recv_sem=recv_sems.at[kk],
                device_id=(right,), device_id_type=pl.DeviceIdType.MESH)
            d.start()
            hop1_r.append(d)
            d = pltpu.make_async_remote_copy(
                src_ref=xown.at[rows, :], dst_ref=comm.at[1, rows, :],
                send_sem=send_sems.at[4 + kk], recv_sem=recv_sems.at[4 + kk],
                device_id=(left,), device_id_type=pl.DeviceIdType.MESH)
            d.start()
            hop1_l.append(d)

        ks = k // KS
        for s in range(KS):
            rows = pl.ds(s * ks, ks)
            cpw = pltpu.make_async_copy(w_hbm.at[rows, :], wtmp,
                                        cp_sems.at[2])
            cpw.start()
            cpw.wait()
            w_bf[rows, :] = wtmp[...].astype(bf16)

        fwd = [None] * 4

        def make_fwd(j):
            src_slot, dst_dev = (0, right) if j < 2 else (1, left)
            rows = pl.ds(j * P, P)
            d = pltpu.make_async_remote_copy(
                src_ref=comm.at[src_slot, rows, :],
                dst_ref=comm.at[2, rows, :],
                send_sem=send_sems.at[8 + j], recv_sem=recv_sems.at[8 + j],
                device_id=(dst_dev,), device_id_type=pl.DeviceIdType.MESH)
            d.start()
            fwd[j] = d

        schedule = []
        for kk in range(4):
            schedule.append((None, None, kk * P, my * m_per + kk * P, None))
        for kk in range(4):
            post_l = (lambda j=kk: make_fwd(j)) if kk < 2 else None
            post_r = (lambda j=kk: make_fwd(j)) if kk >= 2 else None
            schedule.append((hop1_r[kk].wait_recv,
                             comm.at[0, pl.ds(kk * P, P), :], None,
                             left * m_per + kk * P, post_l))
            schedule.append((hop1_l[kk].wait_recv,
                             comm.at[1, pl.ds(kk * P, P), :], None,
                             right * m_per + kk * P, post_r))
        opp = lax.rem(my + 2, N_DEV)
        for j in (0, 2, 1, 3):
            schedule.append(((lambda j=j: fwd[j].wait_recv()),
                             comm.at[2, pl.ds(j * P, P), :], None,
                             opp * m_per + j * P, None))

        st_desc = [None, None]

        def do_gemm(cp_in, own_row, slot, row):
            if cp_in is not None:
                cp_in.wait()
            xv = (xstage[slot, :, :] if own_row is None
                  else xown[pl.ds(own_row, P), :])
            if st_desc[slot] is not None:
                st_desc[slot].wait()
            obuf[slot, :, :] = _gelu(
                jnp.dot(xv, w_bf[...], preferred_element_type=f32))
            st = pltpu.make_async_copy(
                obuf.at[slot], out_hbm.at[pl.ds(row, P), :],
                st_sems.at[slot])
            st.start()
            st_desc[slot] = st

        prev = None
        for i, (wait_fn, src, own_row, row, post) in enumerate(schedule):
            slot = i % 2
            if wait_fn is not None:
                wait_fn()
            cp_in = None
            if src is not None:
                cp_in = pltpu.make_async_copy(src, xstage.at[slot],
                                              cp_sems.at[slot])
                cp_in.start()
            if post is not None:
                post()
            if prev is not None:
                do_gemm(*prev)
            prev = (cp_in, own_row, slot, row)
        do_gemm(*prev)

        for d in hop1_r + hop1_l + fwd:
            d.wait_send()
        st_desc[0].wait()
        st_desc[1].wait()

    out, _ = pl.pallas_call(
        body,
        out_shape=(
            jax.ShapeDtypeStruct((N_DEV * m_per, n_per), f32),
            jax.ShapeDtypeStruct((3, m_per, k), bf16),
        ),
        in_specs=[
            pl.BlockSpec(memory_space=pl.ANY),
            pl.BlockSpec(memory_space=pl.ANY),
        ],
        out_specs=(
            pl.BlockSpec(memory_space=pl.ANY),
            pl.BlockSpec(memory_space=pl.ANY),
        ),
        scratch_shapes=[
            pltpu.VMEM((k, n_per), bf16),
            pltpu.VMEM((m_per, k), bf16),
            pltpu.VMEM((m_per, k), f32),
            pltpu.VMEM((k // KS, n_per), f32),
            pltpu.VMEM((2, P, k), bf16),
            pltpu.VMEM((2, P, n_per), f32),
            pltpu.SemaphoreType.DMA((12,)),
            pltpu.SemaphoreType.DMA((12,)),
            pltpu.SemaphoreType.DMA((3,)),
            pltpu.SemaphoreType.DMA((2,)),
        ],
        compiler_params=pltpu.CompilerParams(
            vmem_limit_bytes=64 * 1024 * 1024,
        ),
    )(x, w_mat)
    return out


# device time: 184619 ns/iter; 1.9565x vs baseline; 1.0205x over previous
import jax
import jax.numpy as jnp
from jax import lax
from jax.experimental import pallas as pl
from jax.experimental.pallas import tpu as pltpu

import os as _os
jax.config.update(
    "jax_compilation_cache_dir",
    _os.path.join(_os.path.dirname(_os.path.abspath(__file__)), "jax_cache"),
)
jax.config.update("jax_persistent_cache_min_compile_time_secs", 0)

N_DEV = 4


def _gelu(y):
    c = 0.7978845608028654
    return 0.5 * y * (1.0 + jnp.tanh(c * (y + 0.044715 * y * y * y)))


def kernel(x, w_mat):
    m_per, k = x.shape
    _, n_per = w_mat.shape
    P = m_per // 4
    f32 = jnp.float32
    bf16 = jnp.bfloat16
    KS = 4

    def body(x_hbm, w_hbm, out_hbm, comm, w_bf, xown, xtmp, wtmp, xstage,
             obuf, send_sems, recv_sems, cp_sems, st_sems):
        my = lax.axis_index("i")
        right = lax.rem(my + 1, N_DEV)
        left = lax.rem(my + N_DEV - 1, N_DEV)
        ks = k // KS

        xcp = []
        for kk in range(2):
            c = pltpu.make_async_copy(
                x_hbm.at[pl.ds(kk * P, P), :], xtmp.at[kk],
                cp_sems.at[kk])
            c.start()
            xcp.append(c)
        wcp = []
        for s in range(2):
            c = pltpu.make_async_copy(
                w_hbm.at[pl.ds(s * ks, ks), :], wtmp.at[s], st_sems.at[s])
            c.start()
            wcp.append(c)

        hop1_r = []
        hop1_l = []
        for kk in range(4):
            rows = pl.ds(kk * P, P)
            xcp[kk].wait()
            xown[rows, :] = xtmp[kk % 2].astype(bf16)
            if kk + 2 < 4:
                c = pltpu.make_async_copy(
                    x_hbm.at[pl.ds((kk + 2) * P, P), :], xtmp.at[kk % 2],
                    cp_sems.at[kk % 2])
                c.start()
                xcp.append(c)
            d = pltpu.make_async_remote_copy(
                src_ref=xown.at[rows, :], dst_ref=comm.at[0, rows, :],
                send_sem=send_sems.at[kk], recv_sem=recv_sems.at[kk],
                device_id=(right,), device_id_type=pl.DeviceIdType.MESH)
            d.start()
            hop1_r.append(d)
            d = pltpu.make_async_remote_copy(
                src_ref=xown.at[rows, :], dst_ref=comm.at[1, rows, :],
                send_sem=send_sems.at[4 + kk], recv_sem=recv_sems.at[4 + kk],
                device_id=(left,), device_id_type=pl.DeviceIdType.MESH)
            d.start()
            hop1_l.append(d)

        for s in range(KS):
            wcp[s].wait()
            w_bf[pl.ds(s * ks, ks), :] = wtmp[s % 2].astype(bf16)
            if s + 2 < KS:
                c = pltpu.make_async_copy(
                    w_hbm.at[pl.ds((s + 2) * ks, ks), :], wtmp.at[s % 2],
                    st_sems.at[s % 2])
                c.start()
                wcp.append(c)

        fwd = [None] * 4

        def make_fwd(j):
            src_slot, dst_dev = (0, right) if j < 2 else (1, left)
            rows = pl.ds(j * P, P)
            d = pltpu.make_async_remote_copy(
                src_ref=comm.at[src_slot, rows, :],
                dst_ref=comm.at[2, rows, :],
                send_sem=send_sems.at[8 + j], recv_sem=recv_sems.at[8 + j],
                device_id=(dst_dev,), device_id_type=pl.DeviceIdType.MESH)
            d.start()
            fwd[j] = d

        schedule = []
        for kk in range(4):
            schedule.append((None, None, kk * P, my * m_per + kk * P, None))
        for kk in range(4):
            post_l = (lambda j=kk: make_fwd(j)) if kk < 2 else None
            post_r = (lambda j=kk: make_fwd(j)) if kk >= 2 else None
            schedule.append((hop1_r[kk].wait_recv,
                             comm.at[0, pl.ds(kk * P, P), :], None,
                             left * m_per + kk * P, post_l))
            schedule.append((hop1_l[kk].wait_recv,
                             comm.at[1, pl.ds(kk * P, P), :], None,
                             right * m_per + kk * P, post_r))
        opp = lax.rem(my + 2, N_DEV)
        for j in (0, 2, 1, 3):
            schedule.append(((lambda j=j: fwd[j].wait_recv()),
                             comm.at[2, pl.ds(j * P, P), :], None,
                             opp * m_per + j * P, None))

        st_desc = [None, None]

        def do_gemm(cp_in, own_row, slot, row):
            if cp_in is not None:
                cp_in.wait()
            xv = (xstage[slot, :, :] if own_row is None
                  else xown[pl.ds(own_row, P), :])
            if st_desc[slot] is not None:
                st_desc[slot].wait()
            obuf[slot, :, :] = _gelu(
                jnp.dot(xv, w_bf[...], preferred_element_type=f32))
            st = pltpu.make_async_copy(
                obuf.at[slot], out_hbm.at[pl.ds(row, P), :],
                st_sems.at[slot])
            st.start()
            st_desc[slot] = st

        prev = None
        for i, (wait_fn, src, own_row, row, post) in enumerate(schedule):
            slot = i % 2
            if wait_fn is not None:
                wait_fn()
            cp_in = None
            if src is not None:
                cp_in = pltpu.make_async_copy(src, xstage.at[slot],
                                              cp_sems.at[slot])
                cp_in.start()
            if post is not None:
                post()
            if prev is not None:
                do_gemm(*prev)
            prev = (cp_in, own_row, slot, row)
        do_gemm(*prev)

        for d in hop1_r + hop1_l + fwd:
            d.wait_send()
        st_desc[0].wait()
        st_desc[1].wait()

    out, _ = pl.pallas_call(
        body,
        out_shape=(
            jax.ShapeDtypeStruct((N_DEV * m_per, n_per), f32),
            jax.ShapeDtypeStruct((3, m_per, k), bf16),
        ),
        in_specs=[
            pl.BlockSpec(memory_space=pl.ANY),
            pl.BlockSpec(memory_space=pl.ANY),
        ],
        out_specs=(
            pl.BlockSpec(memory_space=pl.ANY),
            pl.BlockSpec(memory_space=pl.ANY),
        ),
        scratch_shapes=[
            pltpu.VMEM((k, n_per), bf16),
            pltpu.VMEM((m_per, k), bf16),
            pltpu.VMEM((2, P, k), f32),
            pltpu.VMEM((2, k // KS, n_per), f32),
            pltpu.VMEM((2, P, k), bf16),
            pltpu.VMEM((2, P, n_per), f32),
            pltpu.SemaphoreType.DMA((12,)),
            pltpu.SemaphoreType.DMA((12,)),
            pltpu.SemaphoreType.DMA((3,)),
            pltpu.SemaphoreType.DMA((2,)),
        ],
        compiler_params=pltpu.CompilerParams(
            vmem_limit_bytes=64 * 1024 * 1024,
        ),
    )(x, w_mat)
    return out
